# Initial kernel scaffold; baseline (speedup 1.0000x reference)
import jax
import jax.numpy as jnp
from jax.experimental import pallas as pl


def kernel(x, pos_table):
    raise NotImplementedError("write your pallas kernel here")


# speedup vs baseline: 1.7160x; 1.7160x over previous
import jax
import jax.numpy as jnp
from jax.experimental import pallas as pl

_SEQ_BLOCK = 256


def _add_body(x_ref, pos_ref, o_ref):
    o_ref[...] = x_ref[...] + pos_ref[...][None, :, :]


def kernel(x, pos_table):
    batch, seq_len, d_model = x.shape
    grid = (seq_len // _SEQ_BLOCK,)
    return pl.pallas_call(
        _add_body,
        grid=grid,
        in_specs=[
            pl.BlockSpec((batch, _SEQ_BLOCK, d_model), lambda i: (0, i, 0)),
            pl.BlockSpec((_SEQ_BLOCK, d_model), lambda i: (i, 0)),
        ],
        out_specs=pl.BlockSpec((batch, _SEQ_BLOCK, d_model), lambda i: (0, i, 0)),
        out_shape=jax.ShapeDtypeStruct((batch, seq_len, d_model), x.dtype),
    )(x, pos_table[:seq_len])


# speedup vs baseline: 1.7261x; 1.0059x over previous
import jax
import jax.numpy as jnp
from jax.experimental import pallas as pl

_SEQ_BLOCK = 512


def _add_body(x_ref, pos_ref, o_ref):
    o_ref[...] = x_ref[...] + pos_ref[...][None, :, :]


def kernel(x, pos_table):
    batch, seq_len, d_model = x.shape
    grid = (seq_len // _SEQ_BLOCK,)
    return pl.pallas_call(
        _add_body,
        grid=grid,
        in_specs=[
            pl.BlockSpec((batch, _SEQ_BLOCK, d_model), lambda i: (0, i, 0)),
            pl.BlockSpec((_SEQ_BLOCK, d_model), lambda i: (i, 0)),
        ],
        out_specs=pl.BlockSpec((batch, _SEQ_BLOCK, d_model), lambda i: (0, i, 0)),
        out_shape=jax.ShapeDtypeStruct((batch, seq_len, d_model), x.dtype),
    )(x, pos_table[:seq_len])
